# Initial kernel scaffold; baseline (speedup 1.0000x reference)
import jax
import jax.numpy as jnp
from jax.experimental import pallas as pl


def kernel(input_par_outcome, input_view_ids, query_ids, embedding_table):
    raise NotImplementedError("write your pallas kernel here")


# speedup vs baseline: 1.2924x; 1.2924x over previous
import functools

import jax
import jax.numpy as jnp
from jax import lax
from jax.experimental import pallas as pl
from jax.experimental.pallas import tpu as pltpu
from jax.experimental.pallas import tpu_sc as plsc

_D = 64
_F = 26
_LP1 = 21
_Q = 20
_NC = 2
_NS = 16
_NW = _NC * _NS
_C = 4


def _positions(index_dim, d):
    num_bands = d // 2
    pos = jnp.linspace(-1.0, 1.0, num=index_dim, endpoint=True, dtype=jnp.float32)
    freq = jnp.linspace(1.0, index_dim / 2.0, num=num_bands, endpoint=True,
                        dtype=jnp.float32)
    per_pos = pos[:, None] * freq[None, :]
    return jnp.concatenate(
        [jnp.sin(jnp.pi * per_pos), jnp.cos(jnp.pi * per_pos)], axis=-1)


@functools.cache
def _build(B):
    R1, R2 = B * _LP1, B * _Q
    r1w, r2w = R1 // _NW, R2 // _NW
    n1, n2 = r1w // _C, r2w // _C
    assert r1w % _C == 0 and r2w % _C == 0

    mesh = plsc.VectorSubcoreMesh(
        core_axis_name="c", subcore_axis_name="s",
        num_cores=_NC, num_subcores=_NS)

    @functools.partial(
        pl.kernel,
        mesh=mesh,
        compiler_params=pltpu.CompilerParams(use_tc_tiling_on_sc=False),
        out_type=(
            jax.ShapeDtypeStruct((R1, _D), jnp.float32),
            jax.ShapeDtypeStruct((R2, _D), jnp.float32),
        ),
        scratch_types=[
            pltpu.VMEM((n1, _C * _F), jnp.int32),
            pltpu.VMEM((n2, _C * _F), jnp.int32),
            pltpu.VMEM((_LP1, _D), jnp.float32),
            pltpu.VMEM((_C * _F, _D), jnp.float32),
            pltpu.VMEM((_C, _D), jnp.float32),
            pltpu.SemaphoreType.DMA,
        ],
    )
    def k(table, ids1, ids2, pos, out1, out2,
          ids1_v, ids2_v, pos_v, rows_v, outb_v, sem):
        wid = lax.axis_index("s") * _NC + lax.axis_index("c")
        b1 = wid * r1w
        b2 = wid * r2w
        pltpu.sync_copy(ids1.at[wid], ids1_v)
        pltpu.sync_copy(ids2.at[wid], ids2_v)
        pltpu.sync_copy(pos, pos_v)

        def reduce_rows(j, base_rows):
            accs = [rows_v[j * _F, pl.ds(col * 16, 16)] for col in range(4)]
            for f in range(1, _F):
                for col in range(4):
                    accs[col] = accs[col] + rows_v[j * _F + f, pl.ds(col * 16, 16)]
            return accs

        def body1(ci, carry):
            pltpu.async_copy(table.at[ids1_v.at[ci]], rows_v, sem).wait()
            r0 = b1 + ci * _C
            for j in range(_C):
                lrow = lax.rem(r0 + j, _LP1)
                accs = reduce_rows(j, None)
                for col in range(4):
                    outb_v[j, pl.ds(col * 16, 16)] = (
                        accs[col] + pos_v[lrow, pl.ds(col * 16, 16)])
            pltpu.sync_copy(outb_v, out1.at[pl.ds(r0, _C)])
            return carry

        lax.fori_loop(0, n1, body1, 0)

        def body2(ci, carry):
            pltpu.async_copy(table.at[ids2_v.at[ci]], rows_v, sem).wait()
            r0 = b2 + ci * _C
            for j in range(_C):
                accs = reduce_rows(j, None)
                for col in range(4):
                    outb_v[j, pl.ds(col * 16, 16)] = accs[col]
            pltpu.sync_copy(outb_v, out2.at[pl.ds(r0, _C)])
            return carry

        lax.fori_loop(0, n2, body2, 0)

    return k


def kernel(input_par_outcome, input_view_ids, query_ids, embedding_table):
    B = input_par_outcome.shape[0]
    ids1 = jnp.concatenate(
        [input_par_outcome[:, None, :], input_view_ids], axis=1)
    ids1 = ids1.reshape(_NW, (B * _LP1 // _NW) // _C, _C * _F)
    ids2 = query_ids.reshape(_NW, (B * _Q // _NW) // _C, _C * _F)
    pos = _positions(_LP1, _D)
    out1, out2 = _build(B)(embedding_table, ids1, ids2, pos)
    return (out1.reshape(B, _LP1, _D), None, out2.reshape(B, _Q, _D))


# speedup vs baseline: 1.4347x; 1.1101x over previous
import functools

import jax
import jax.numpy as jnp
from jax import lax
from jax.experimental import pallas as pl
from jax.experimental.pallas import tpu as pltpu
from jax.experimental.pallas import tpu_sc as plsc

_D = 64
_F = 26
_LP1 = 21
_Q = 20
_NC = 2
_NS = 16
_NW = _NC * _NS
_C = 4
_NBUF = 4


def _positions(index_dim, d):
    num_bands = d // 2
    pos = jnp.linspace(-1.0, 1.0, num=index_dim, endpoint=True, dtype=jnp.float32)
    freq = jnp.linspace(1.0, index_dim / 2.0, num=num_bands, endpoint=True,
                        dtype=jnp.float32)
    per_pos = pos[:, None] * freq[None, :]
    return jnp.concatenate(
        [jnp.sin(jnp.pi * per_pos), jnp.cos(jnp.pi * per_pos)], axis=-1)


@functools.cache
def _build(B):
    R1, R2 = B * _LP1, B * _Q
    r1w, r2w = R1 // _NW, R2 // _NW
    n1, n2 = r1w // _C, r2w // _C
    assert r1w % _C == 0 and r2w % _C == 0
    assert n1 % _NBUF == 0 and n2 % _NBUF == 0

    mesh = plsc.VectorSubcoreMesh(
        core_axis_name="c", subcore_axis_name="s",
        num_cores=_NC, num_subcores=_NS)

    @functools.partial(
        pl.kernel,
        mesh=mesh,
        compiler_params=pltpu.CompilerParams(use_tc_tiling_on_sc=False),
        out_type=(
            jax.ShapeDtypeStruct((R1, _D), jnp.float32),
            jax.ShapeDtypeStruct((R2, _D), jnp.float32),
        ),
        scratch_types=[
            pltpu.VMEM((n1, _C * _F), jnp.int32),
            pltpu.VMEM((n2, _C * _F), jnp.int32),
            pltpu.VMEM((_LP1, _D), jnp.float32),
            pltpu.VMEM((_NBUF, _C * _F, _D), jnp.float32),
            pltpu.VMEM((_NBUF, _C, _D), jnp.float32),
            pltpu.SemaphoreType.DMA((_NBUF,)),
            pltpu.SemaphoreType.DMA((_NBUF,)),
        ],
    )
    def k(table, ids1, ids2, pos, out1, out2,
          ids1_v, ids2_v, pos_v, rows_v, outs_v, gsem, ssem):
        wid = lax.axis_index("s") * _NC + lax.axis_index("c")
        pltpu.sync_copy(ids1.at[wid], ids1_v)
        pltpu.sync_copy(ids2.at[wid], ids2_v)
        pltpu.sync_copy(pos, pos_v)

        def gather(ids_v, ci, b):
            return pltpu.make_async_copy(
                table.at[ids_v.at[ci]], rows_v.at[b], gsem.at[b])

        def store(out_hbm, r0, b):
            return pltpu.make_async_copy(
                outs_v.at[b], out_hbm.at[pl.ds(r0, _C)], ssem.at[b])

        def phase(ids_v, nchunks, out_hbm, base, add_pos):
            for b in range(_NBUF):
                gather(ids_v, b, b).start()

            def group_body(g, carry):
                for b in range(_NBUF):
                    ci = g * _NBUF + b
                    r0 = base + ci * _C
                    gather(ids_v, ci, b).wait()

                    @pl.when(g >= 1)
                    def _():
                        store(out_hbm, base, b).wait()

                    for j in range(_C):
                        accs = [rows_v[b, j * _F, pl.ds(col * 16, 16)]
                                for col in range(4)]
                        for f in range(1, _F):
                            for col in range(4):
                                accs[col] = accs[col] + rows_v[
                                    b, j * _F + f, pl.ds(col * 16, 16)]
                        if add_pos:
                            lrow = lax.rem(r0 + j, _LP1)
                            for col in range(4):
                                accs[col] = accs[col] + pos_v[
                                    lrow, pl.ds(col * 16, 16)]
                        for col in range(4):
                            outs_v[b, j, pl.ds(col * 16, 16)] = accs[col]
                    store(out_hbm, r0, b).start()

                    @pl.when(ci + _NBUF < nchunks)
                    def _():
                        gather(ids_v, ci + _NBUF, b).start()
                return carry

            lax.fori_loop(0, nchunks // _NBUF, group_body, 0)
            for b in range(_NBUF):
                store(out_hbm, base, b).wait()

        phase(ids1_v, n1, out1, wid * r1w, True)
        phase(ids2_v, n2, out2, wid * r2w, False)

    return k


def kernel(input_par_outcome, input_view_ids, query_ids, embedding_table):
    B = input_par_outcome.shape[0]
    ids1 = jnp.concatenate(
        [input_par_outcome[:, None, :], input_view_ids], axis=1)
    ids1 = ids1.reshape(_NW, (B * _LP1 // _NW) // _C, _C * _F)
    ids2 = query_ids.reshape(_NW, (B * _Q // _NW) // _C, _C * _F)
    pos = _positions(_LP1, _D)
    out1, out2 = _build(B)(embedding_table, ids1, ids2, pos)
    return (out1.reshape(B, _LP1, _D), None, out2.reshape(B, _Q, _D))


# speedup vs baseline: 1.5527x; 1.0822x over previous
import functools

import jax
import jax.numpy as jnp
from jax import lax
from jax.experimental import pallas as pl
from jax.experimental.pallas import tpu as pltpu
from jax.experimental.pallas import tpu_sc as plsc

_D = 64
_F = 26
_L = 20
_LP1 = _L + 1
_Q = 20
_NC = 2
_NS = 16
_NW = _NC * _NS
_C = 4
_NBUF = 2


def _positions(index_dim, d):
    num_bands = d // 2
    pos = jnp.linspace(-1.0, 1.0, num=index_dim, endpoint=True, dtype=jnp.float32)
    freq = jnp.linspace(1.0, index_dim / 2.0, num=num_bands, endpoint=True,
                        dtype=jnp.float32)
    per_pos = pos[:, None] * freq[None, :]
    return jnp.concatenate(
        [jnp.sin(jnp.pi * per_pos), jnp.cos(jnp.pi * per_pos)], axis=-1)


@functools.cache
def _build(B):
    npar = (B // _NW) // _C
    nv = (B * _L // _NW) // _C
    nq = (B * _Q // _NW) // _C
    rvw = B * _L // _NW
    rqw = B * _Q // _NW
    rpw = B // _NW
    assert B % (_NW * _C) == 0 and nv % _NBUF == 0 and nq % _NBUF == 0
    assert npar % _NBUF == 0

    mesh = plsc.VectorSubcoreMesh(
        core_axis_name="c", subcore_axis_name="s",
        num_cores=_NC, num_subcores=_NS)

    @functools.partial(
        pl.kernel,
        mesh=mesh,
        compiler_params=pltpu.CompilerParams(use_tc_tiling_on_sc=False),
        out_type=(
            jax.ShapeDtypeStruct((B * _LP1, _D), jnp.float32),
            jax.ShapeDtypeStruct((B * _Q, _D), jnp.float32),
        ),
        scratch_types=[
            pltpu.VMEM((npar, _C * _F), jnp.int32),
            pltpu.VMEM((nv, _C * _F), jnp.int32),
            pltpu.VMEM((nq, _C * _F), jnp.int32),
            pltpu.VMEM((_LP1, _D), jnp.float32),
            pltpu.VMEM((_NBUF, _C * _F, _D), jnp.float32),
            pltpu.VMEM((_NBUF, _C, _D), jnp.float32),
            pltpu.SemaphoreType.DMA((_NBUF,)),
            pltpu.SemaphoreType.DMA((_NBUF,)),
        ],
    )
    def k(table, idsp, idsv, idsq, pos, out1, out2,
          idsp_v, idsv_v, idsq_v, pos_v, rows_v, outs_v, gsem, ssem):
        wid = lax.axis_index("s") * _NC + lax.axis_index("c")
        pltpu.sync_copy(idsp.at[wid], idsp_v)
        pltpu.sync_copy(idsv.at[wid], idsv_v)
        pltpu.sync_copy(idsq.at[wid], idsq_v)
        pltpu.sync_copy(pos, pos_v)

        def gather(ids_v, ci, b):
            return pltpu.make_async_copy(
                table.at[ids_v.at[ci]], rows_v.at[b], gsem.at[b])

        def reduce_chunk(b, lrow_of_j, out_row_of_j):
            for j in range(_C):
                accs = [rows_v[b, j * _F, pl.ds(col * 16, 16)]
                        for col in range(4)]
                for f in range(1, _F):
                    for col in range(4):
                        accs[col] = accs[col] + rows_v[
                            b, j * _F + f, pl.ds(col * 16, 16)]
                if lrow_of_j is not None:
                    lrow = lrow_of_j(j)
                    for col in range(4):
                        accs[col] = accs[col] + pos_v[lrow, pl.ds(col * 16, 16)]
                for col in range(4):
                    outs_v[b, out_row_of_j(j), pl.ds(col * 16, 16)] = accs[col]

        def phase(ids_v, nchunks, out_hbm, row0_of_ci, lrow_of, store_rows):
            def store(ci, b):
                if store_rows == _C:
                    return [pltpu.make_async_copy(
                        outs_v.at[b], out_hbm.at[pl.ds(row0_of_ci(ci), _C)],
                        ssem.at[b])]
                return [pltpu.make_async_copy(
                    outs_v.at[b].at[pl.ds(j, 1)],
                    out_hbm.at[pl.ds(row0_of_ci(ci) + j * _LP1, 1)],
                    ssem.at[b]) for j in range(_C)]

            for b in range(_NBUF):
                gather(ids_v, b, b).start()

            def group_body(g, carry):
                for b in range(_NBUF):
                    ci = g * _NBUF + b
                    gather(ids_v, ci, b).wait()

                    @pl.when(g >= 1)
                    def _():
                        for s in store(0, b):
                            s.wait()

                    reduce_chunk(b, lrow_of and (lambda j: lrow_of(ci, j)),
                                 lambda j: j)
                    for s in store(ci, b):
                        s.start()

                    @pl.when(ci + _NBUF < nchunks)
                    def _():
                        gather(ids_v, ci + _NBUF, b).start()
                return carry

            lax.fori_loop(0, nchunks // _NBUF, group_body, 0)
            for b in range(_NBUF):
                for s in store(0, b):
                    s.wait()

        phase(idsp_v, npar, out1,
              lambda ci: (wid * rpw + ci * _C) * _LP1,
              lambda ci, j: 0, 1)
        phase(idsv_v, nv, out1,
              lambda ci: (lambda v0: v0 + v0 // _L + 1)(wid * rvw + ci * _C),
              lambda ci, j: 1 + lax.rem(wid * rvw + ci * _C + j, _L), _C)
        phase(idsq_v, nq, out2,
              lambda ci: wid * rqw + ci * _C, None, _C)

    return k


def kernel(input_par_outcome, input_view_ids, query_ids, embedding_table):
    B = input_par_outcome.shape[0]
    idsp = input_par_outcome.reshape(_NW, (B // _NW) // _C, _C * _F)
    idsv = input_view_ids.reshape(_NW, (B * _L // _NW) // _C, _C * _F)
    idsq = query_ids.reshape(_NW, (B * _Q // _NW) // _C, _C * _F)
    pos = _positions(_LP1, _D)
    out1, out2 = _build(B)(embedding_table, idsp, idsv, idsq, pos)
    return (out1.reshape(B, _LP1, _D), None, out2.reshape(B, _Q, _D))


# speedup vs baseline: 1.9041x; 1.2263x over previous
import functools

import jax
import jax.numpy as jnp
from jax import lax
from jax.experimental import pallas as pl
from jax.experimental.pallas import tpu as pltpu
from jax.experimental.pallas import tpu_sc as plsc

_D = 64
_F = 26
_L = 20
_LP1 = _L + 1
_Q = 20
_NC = 2
_NS = 16
_NW = _NC * _NS
_C = 4
_NBUF = 6
_NSB = 4


def _positions(index_dim, d):
    num_bands = d // 2
    pos = jnp.linspace(-1.0, 1.0, num=index_dim, endpoint=True, dtype=jnp.float32)
    freq = jnp.linspace(1.0, index_dim / 2.0, num=num_bands, endpoint=True,
                        dtype=jnp.float32)
    per_pos = pos[:, None] * freq[None, :]
    return jnp.concatenate(
        [jnp.sin(jnp.pi * per_pos), jnp.cos(jnp.pi * per_pos)], axis=-1)


@functools.cache
def _build(B):
    npar = (B // _NW) // _C
    nv = (B * _L // _NW) // _C
    nq = (B * _Q // _NW) // _C
    rvw = B * _L // _NW
    rqw = B * _Q // _NW
    rpw = B // _NW
    assert B % (_NW * _C) == 0

    mesh = plsc.VectorSubcoreMesh(
        core_axis_name="c", subcore_axis_name="s",
        num_cores=_NC, num_subcores=_NS)

    @functools.partial(
        pl.kernel,
        mesh=mesh,
        compiler_params=pltpu.CompilerParams(use_tc_tiling_on_sc=False),
        out_type=(
            jax.ShapeDtypeStruct((B * _LP1, _D), jnp.float32),
            jax.ShapeDtypeStruct((B * _Q, _D), jnp.float32),
        ),
        scratch_types=[
            pltpu.VMEM((npar, _C * _F), jnp.int32),
            pltpu.VMEM((nv, _C * _F), jnp.int32),
            pltpu.VMEM((nq, _C * _F), jnp.int32),
            pltpu.VMEM((_LP1, _D), jnp.float32),
            pltpu.VMEM((_NBUF, _C * _F, _D), jnp.float32),
            pltpu.VMEM((_NSB, _C, _D), jnp.float32),
            pltpu.SemaphoreType.DMA((_NBUF,)),
            pltpu.SemaphoreType.DMA((_NSB,)),
        ],
    )
    def k(table, idsp, idsv, idsq, pos, out1, out2,
          idsp_v, idsv_v, idsq_v, pos_v, rows_v, outs_v, gsem, ssem):
        wid = lax.axis_index("s") * _NC + lax.axis_index("c")
        pltpu.sync_copy(idsp.at[wid], idsp_v)
        pltpu.sync_copy(idsv.at[wid], idsv_v)
        pltpu.sync_copy(idsq.at[wid], idsq_v)
        pltpu.sync_copy(pos, pos_v)

        def gather(ids_v, ci, b):
            return pltpu.make_async_copy(
                table.at[ids_v.at[ci]], rows_v.at[b], gsem.at[b])

        def reduce_chunk(b, s, lrow_of_j):
            for j in range(_C):
                accs = [rows_v[b, j * _F, pl.ds(col * 16, 16)]
                        for col in range(4)]
                for f in range(1, _F):
                    for col in range(4):
                        accs[col] = accs[col] + rows_v[
                            b, j * _F + f, pl.ds(col * 16, 16)]
                if lrow_of_j is not None:
                    lrow = lrow_of_j(j)
                    for col in range(4):
                        accs[col] = accs[col] + pos_v[lrow, pl.ds(col * 16, 16)]
                for col in range(4):
                    outs_v[s, j, pl.ds(col * 16, 16)] = accs[col]

        def phase(ids_v, nchunks, out_hbm, row0_of_ci, lrow_of, store_rows):
            def store(ci, s):
                if store_rows == _C:
                    return [pltpu.make_async_copy(
                        outs_v.at[s], out_hbm.at[pl.ds(row0_of_ci(ci), _C)],
                        ssem.at[s])]
                return [pltpu.make_async_copy(
                    outs_v.at[s].at[pl.ds(j, 1)],
                    out_hbm.at[pl.ds(row0_of_ci(ci) + j * _LP1, 1)],
                    ssem.at[s]) for j in range(_C)]

            for b in range(min(_NBUF, nchunks)):
                gather(ids_v, b, b).start()

            def chunk_body(ci, carry):
                b = lax.rem(ci, _NBUF)
                s = lax.rem(ci, _NSB)
                gather(ids_v, ci, b).wait()

                @pl.when(ci >= _NSB)
                def _():
                    for d in store(0, s):
                        d.wait()

                reduce_chunk(b, s, lrow_of and (lambda j: lrow_of(ci, j)))
                for d in store(ci, s):
                    d.start()

                @pl.when(ci + _NBUF < nchunks)
                def _():
                    gather(ids_v, ci + _NBUF, b).start()
                return carry

            lax.fori_loop(0, nchunks, chunk_body, 0)
            for s in range(min(_NSB, nchunks)):
                for d in store(0, s):
                    d.wait()

        phase(idsp_v, npar, out1,
              lambda ci: (wid * rpw + ci * _C) * _LP1,
              lambda ci, j: 0, 1)
        phase(idsv_v, nv, out1,
              lambda ci: (lambda v0: v0 + v0 // _L + 1)(wid * rvw + ci * _C),
              lambda ci, j: 1 + lax.rem(wid * rvw + ci * _C + j, _L), _C)
        phase(idsq_v, nq, out2,
              lambda ci: wid * rqw + ci * _C, None, _C)

    return k


def kernel(input_par_outcome, input_view_ids, query_ids, embedding_table):
    B = input_par_outcome.shape[0]
    idsp = input_par_outcome.reshape(_NW, (B // _NW) // _C, _C * _F)
    idsv = input_view_ids.reshape(_NW, (B * _L // _NW) // _C, _C * _F)
    idsq = query_ids.reshape(_NW, (B * _Q // _NW) // _C, _C * _F)
    pos = _positions(_LP1, _D)
    out1, out2 = _build(B)(embedding_table, idsp, idsv, idsq, pos)
    return (out1.reshape(B, _LP1, _D), None, out2.reshape(B, _Q, _D))


# speedup vs baseline: 1.9041x; 1.0000x over previous
import functools

import jax
import jax.numpy as jnp
from jax import lax
from jax.experimental import pallas as pl
from jax.experimental.pallas import tpu as pltpu
from jax.experimental.pallas import tpu_sc as plsc

_D = 64
_F = 26
_L = 20
_LP1 = _L + 1
_Q = 20
_NC = 2
_NS = 16
_NW = _NC * _NS
_C = 4
_NBUF = 8
_NSB = 6


def _positions(index_dim, d):
    num_bands = d // 2
    pos = jnp.linspace(-1.0, 1.0, num=index_dim, endpoint=True, dtype=jnp.float32)
    freq = jnp.linspace(1.0, index_dim / 2.0, num=num_bands, endpoint=True,
                        dtype=jnp.float32)
    per_pos = pos[:, None] * freq[None, :]
    return jnp.concatenate(
        [jnp.sin(jnp.pi * per_pos), jnp.cos(jnp.pi * per_pos)], axis=-1)


@functools.cache
def _build(B):
    npar = (B // _NW) // _C
    nv = (B * _L // _NW) // _C
    nq = (B * _Q // _NW) // _C
    rvw = B * _L // _NW
    rqw = B * _Q // _NW
    rpw = B // _NW
    assert B % (_NW * _C) == 0

    mesh = plsc.VectorSubcoreMesh(
        core_axis_name="c", subcore_axis_name="s",
        num_cores=_NC, num_subcores=_NS)

    @functools.partial(
        pl.kernel,
        mesh=mesh,
        compiler_params=pltpu.CompilerParams(use_tc_tiling_on_sc=False),
        out_type=(
            jax.ShapeDtypeStruct((B * _LP1, _D), jnp.float32),
            jax.ShapeDtypeStruct((B * _Q, _D), jnp.float32),
        ),
        scratch_types=[
            pltpu.VMEM((npar, _C * _F), jnp.int32),
            pltpu.VMEM((nv, _C * _F), jnp.int32),
            pltpu.VMEM((nq, _C * _F), jnp.int32),
            pltpu.VMEM((_LP1, _D), jnp.float32),
            pltpu.VMEM((_NBUF, _C * _F, _D), jnp.float32),
            pltpu.VMEM((_NSB, _C, _D), jnp.float32),
            pltpu.SemaphoreType.DMA((_NBUF,)),
            pltpu.SemaphoreType.DMA((_NSB,)),
        ],
    )
    def k(table, idsp, idsv, idsq, pos, out1, out2,
          idsp_v, idsv_v, idsq_v, pos_v, rows_v, outs_v, gsem, ssem):
        wid = lax.axis_index("s") * _NC + lax.axis_index("c")
        pltpu.sync_copy(idsp.at[wid], idsp_v)
        pltpu.sync_copy(idsv.at[wid], idsv_v)
        pltpu.sync_copy(idsq.at[wid], idsq_v)
        pltpu.sync_copy(pos, pos_v)

        def gather(ids_v, ci, b):
            return pltpu.make_async_copy(
                table.at[ids_v.at[ci]], rows_v.at[b], gsem.at[b])

        def reduce_chunk(b, s, lrow_of_j):
            for j in range(_C):
                accs = [rows_v[b, j * _F, pl.ds(col * 16, 16)]
                        for col in range(4)]
                for f in range(1, _F):
                    for col in range(4):
                        accs[col] = accs[col] + rows_v[
                            b, j * _F + f, pl.ds(col * 16, 16)]
                if lrow_of_j is not None:
                    lrow = lrow_of_j(j)
                    for col in range(4):
                        accs[col] = accs[col] + pos_v[lrow, pl.ds(col * 16, 16)]
                for col in range(4):
                    outs_v[s, j, pl.ds(col * 16, 16)] = accs[col]

        def phase(ids_v, nchunks, out_hbm, row0_of_ci, lrow_of, store_rows):
            def store(ci, s):
                if store_rows == _C:
                    return [pltpu.make_async_copy(
                        outs_v.at[s], out_hbm.at[pl.ds(row0_of_ci(ci), _C)],
                        ssem.at[s])]
                return [pltpu.make_async_copy(
                    outs_v.at[s].at[pl.ds(j, 1)],
                    out_hbm.at[pl.ds(row0_of_ci(ci) + j * _LP1, 1)],
                    ssem.at[s]) for j in range(_C)]

            for b in range(min(_NBUF, nchunks)):
                gather(ids_v, b, b).start()

            def chunk_body(ci, carry):
                b = lax.rem(ci, _NBUF)
                s = lax.rem(ci, _NSB)
                gather(ids_v, ci, b).wait()

                @pl.when(ci >= _NSB)
                def _():
                    for d in store(0, s):
                        d.wait()

                reduce_chunk(b, s, lrow_of and (lambda j: lrow_of(ci, j)))
                for d in store(ci, s):
                    d.start()

                @pl.when(ci + _NBUF < nchunks)
                def _():
                    gather(ids_v, ci + _NBUF, b).start()
                return carry

            lax.fori_loop(0, nchunks, chunk_body, 0)
            for s in range(min(_NSB, nchunks)):
                for d in store(0, s):
                    d.wait()

        phase(idsp_v, npar, out1,
              lambda ci: (wid * rpw + ci * _C) * _LP1,
              lambda ci, j: 0, 1)
        phase(idsv_v, nv, out1,
              lambda ci: (lambda v0: v0 + v0 // _L + 1)(wid * rvw + ci * _C),
              lambda ci, j: 1 + lax.rem(wid * rvw + ci * _C + j, _L), _C)
        phase(idsq_v, nq, out2,
              lambda ci: wid * rqw + ci * _C, None, _C)

    return k


def kernel(input_par_outcome, input_view_ids, query_ids, embedding_table):
    B = input_par_outcome.shape[0]
    idsp = input_par_outcome.reshape(_NW, (B // _NW) // _C, _C * _F)
    idsv = input_view_ids.reshape(_NW, (B * _L // _NW) // _C, _C * _F)
    idsq = query_ids.reshape(_NW, (B * _Q // _NW) // _C, _C * _F)
    pos = _positions(_LP1, _D)
    out1, out2 = _build(B)(embedding_table, idsp, idsv, idsq, pos)
    return (out1.reshape(B, _LP1, _D), None, out2.reshape(B, _Q, _D))


# speedup vs baseline: 1.9083x; 1.0022x over previous
import functools

import jax
import jax.numpy as jnp
from jax import lax
from jax.experimental import pallas as pl
from jax.experimental.pallas import tpu as pltpu
from jax.experimental.pallas import tpu_sc as plsc

_D = 64
_F = 26
_L = 20
_LP1 = _L + 1
_Q = 20
_NC = 2
_NS = 16
_NW = _NC * _NS
_C = 4
_NBUF = 6
_NSB = 4


def _positions(index_dim, d):
    num_bands = d // 2
    pos = jnp.linspace(-1.0, 1.0, num=index_dim, endpoint=True, dtype=jnp.float32)
    freq = jnp.linspace(1.0, index_dim / 2.0, num=num_bands, endpoint=True,
                        dtype=jnp.float32)
    per_pos = pos[:, None] * freq[None, :]
    return jnp.concatenate(
        [jnp.sin(jnp.pi * per_pos), jnp.cos(jnp.pi * per_pos)], axis=-1)


@functools.cache
def _build(B):
    npar = (B // _NW) // _C
    nv = (B * _L // _NW) // _C
    nq = (B * _Q // _NW) // _C
    rvw = B * _L // _NW
    rqw = B * _Q // _NW
    rpw = B // _NW
    assert B % (_NW * _C) == 0

    mesh = plsc.VectorSubcoreMesh(
        core_axis_name="c", subcore_axis_name="s",
        num_cores=_NC, num_subcores=_NS)

    @functools.partial(
        pl.kernel,
        mesh=mesh,
        compiler_params=pltpu.CompilerParams(use_tc_tiling_on_sc=False),
        out_type=(
            jax.ShapeDtypeStruct((B * _LP1, _D), jnp.float32),
            jax.ShapeDtypeStruct((B * _Q, _D), jnp.float32),
        ),
        scratch_types=[
            pltpu.VMEM((npar, _C * _F), jnp.int32),
            pltpu.VMEM((nv, _C * _F), jnp.int32),
            pltpu.VMEM((nq, _C * _F), jnp.int32),
            pltpu.VMEM((_LP1, _D), jnp.float32),
            pltpu.VMEM((_NBUF, _C * _F, _D), jnp.float32),
            pltpu.VMEM((_NSB, _C, _D), jnp.float32),
            pltpu.SemaphoreType.DMA((_NBUF,)),
            pltpu.SemaphoreType.DMA((_NSB,)),
        ],
    )
    def k(table, idsp, idsv, idsq, pos, out1, out2,
          idsp_v, idsv_v, idsq_v, pos_v, rows_v, outs_v, gsem, ssem):
        wid = lax.axis_index("s") * _NC + lax.axis_index("c")
        pltpu.sync_copy(idsp.at[wid], idsp_v)
        pltpu.sync_copy(idsv.at[wid], idsv_v)
        pltpu.sync_copy(idsq.at[wid], idsq_v)
        pltpu.sync_copy(pos, pos_v)

        def gather(ids_v, ci, b):
            return pltpu.make_async_copy(
                table.at[ids_v.at[ci]], rows_v.at[b], gsem.at[b])

        def reduce_chunk(b, s, lrow_of_j):
            for j in range(_C):
                accs = [rows_v[b, j * _F, pl.ds(col * 16, 16)]
                        for col in range(4)]
                for f in range(1, _F):
                    for col in range(4):
                        accs[col] = accs[col] + rows_v[
                            b, j * _F + f, pl.ds(col * 16, 16)]
                if lrow_of_j is not None:
                    lrow = lrow_of_j(j)
                    for col in range(4):
                        accs[col] = accs[col] + pos_v[lrow, pl.ds(col * 16, 16)]
                for col in range(4):
                    outs_v[s, j, pl.ds(col * 16, 16)] = accs[col]

        def phase(ids_v, nchunks, out_hbm, row0_of_ci, lrow_of, store_rows):
            def store(ci, s):
                if store_rows == _C:
                    return [pltpu.make_async_copy(
                        outs_v.at[s], out_hbm.at[pl.ds(row0_of_ci(ci), _C)],
                        ssem.at[s])]
                return [pltpu.make_async_copy(
                    outs_v.at[s].at[pl.ds(j, 1)],
                    out_hbm.at[pl.ds(row0_of_ci(ci) + j * _LP1, 1)],
                    ssem.at[s]) for j in range(_C)]

            for b in range(min(_NBUF, nchunks)):
                gather(ids_v, b, b).start()

            def chunk_body(ci, carry):
                b = lax.rem(ci, _NBUF)
                s = lax.rem(ci, _NSB)
                gather(ids_v, ci, b).wait()

                @pl.when(ci >= _NSB)
                def _():
                    for d in store(0, s):
                        d.wait()

                reduce_chunk(b, s, lrow_of and (lambda j: lrow_of(ci, j)))
                for d in store(ci, s):
                    d.start()

                @pl.when(ci + _NBUF < nchunks)
                def _():
                    gather(ids_v, ci + _NBUF, b).start()
                return carry

            lax.fori_loop(0, nchunks, chunk_body, 0)
            for s in range(min(_NSB, nchunks)):
                for d in store(0, s):
                    d.wait()

        phase(idsp_v, npar, out1,
              lambda ci: (wid * rpw + ci * _C) * _LP1,
              lambda ci, j: 0, 1)
        phase(idsv_v, nv, out1,
              lambda ci: (lambda v0: v0 + v0 // _L + 1)(wid * rvw + ci * _C),
              lambda ci, j: 1 + lax.rem(wid * rvw + ci * _C + j, _L), _C)
        phase(idsq_v, nq, out2,
              lambda ci: wid * rqw + ci * _C, None, _C)

    return k


def kernel(input_par_outcome, input_view_ids, query_ids, embedding_table):
    B = input_par_outcome.shape[0]
    idsp = input_par_outcome.reshape(_NW, (B // _NW) // _C, _C * _F)
    idsv = input_view_ids.reshape(_NW, (B * _L // _NW) // _C, _C * _F)
    idsq = query_ids.reshape(_NW, (B * _Q // _NW) // _C, _C * _F)
    pos = _positions(_LP1, _D)
    out1, out2 = _build(B)(embedding_table, idsp, idsv, idsq, pos)
    return (out1.reshape(B, _LP1, _D), None, out2.reshape(B, _Q, _D))
